# baseline (speedup 1.0000x reference)
import functools

import jax
import jax.numpy as jnp
from jax import lax
from jax.experimental import pallas as pl
from jax.experimental.pallas import tpu as pltpu
from jax.experimental.pallas import tpu_sc as plsc

N = 100000
NPAD = 100352
E = 3200000
EPAD = 3276800
NW = 32
EW = EPAD // NW
BLK = 640
NBLK = EW // BLK
ROWS_PER_W = EW // 128
NGRAPH = 128
NODE_BLK = 1024
NBLK_FAST = 303
NBLK_SLOW = 17
NGRID = NPAD // NODE_BLK
TILE_ROWS = NPAD // 16


def _sc_counts(dst_flat):
    mesh = plsc.VectorSubcoreMesh(core_axis_name="c", subcore_axis_name="s")

    @functools.partial(
        pl.kernel,
        out_type=jax.ShapeDtypeStruct((NW, NPAD), jnp.float32),
        scratch_types=[
            pltpu.VMEM((BLK,), jnp.int32),
            pltpu.VMEM((NPAD,), jnp.float32),
        ],
        mesh=mesh,
        compiler_params=pltpu.CompilerParams(use_tc_tiling_on_sc=False,
                                             needs_layout_passes=False),
    )
    def k(dst_h, out_h, dstbuf, cnt):
        c = lax.axis_index("c")
        s = lax.axis_index("s")
        w = s * 2 + c

        def zero_body(j, carry):
            cnt[pl.ds(j * 16, 16)] = jnp.zeros((16,), jnp.float32)
            return carry

        lax.fori_loop(0, NPAD // 16, zero_body, 0)

        ones = jnp.ones((16,), jnp.float32)

        def block_body(b, carry):
            base = w * EW + b * BLK
            pltpu.sync_copy(dst_h.at[pl.ds(base, BLK)], dstbuf)

            def inner(k2, carry2):
                idx = dstbuf[pl.ds(k2 * 16, 16)]
                plsc.addupdate_scatter(cnt, [idx], ones)
                return carry2

            lax.fori_loop(0, BLK // 16, inner, 0)
            return carry

        lax.fori_loop(0, NBLK, block_body, 0)
        pltpu.sync_copy(cnt, out_h.at[w])

    return k(dst_flat)


def _sc_segsum32bf(src_flat, dst_flat, table, zeros32):
    mesh = plsc.VectorSubcoreMesh(core_axis_name="c", subcore_axis_name="s")

    @functools.partial(
        pl.kernel,
        out_type=jax.ShapeDtypeStruct((2, NPAD, 32), jnp.bfloat16),
        scratch_types=[
            pltpu.VMEM((2, BLK), jnp.int32),
            pltpu.VMEM((2, BLK), jnp.int32),
            pltpu.VMEM((2, BLK, 32), jnp.bfloat16),
            pltpu.VMEM_SHARED((NPAD, 32), jnp.bfloat16),
            pltpu.SemaphoreType.DMA,
            pltpu.SemaphoreType.DMA,
            pltpu.SemaphoreType.DMA,
        ],
        mesh=mesh,
        compiler_params=pltpu.CompilerParams(use_tc_tiling_on_sc=False),
    )
    def k(src_h, dst_h, tab_h, z_h, out_h, srcv, dstv, rows, shared,
          sem_i, sem_g, sem_s):
        c = lax.axis_index("c")
        s = lax.axis_index("s")

        pltpu.sync_copy(z_h.at[pl.ds(s * TILE_ROWS, TILE_ROWS)],
                        shared.at[pl.ds(s * TILE_ROWS, TILE_ROWS)])
        plsc.subcore_barrier()

        nblk = jnp.where(c == 0, NBLK_FAST, NBLK_SLOW)
        sc_base = jnp.where(c == 0, 0, NBLK_FAST * 16 * BLK)
        tile_base = sc_base + s * nblk * BLK

        def fire_idx(buf, b):
            base = tile_base + b * BLK
            pltpu.async_copy(src_h.at[pl.ds(base, BLK)], srcv.at[buf],
                             sem_i)
            pltpu.async_copy(dst_h.at[pl.ds(base, BLK)], dstv.at[buf],
                             sem_i)

        def drain_idx(buf, b):
            base = tile_base + b * BLK
            pltpu.make_async_copy(src_h.at[pl.ds(base, BLK)],
                                  srcv.at[buf], sem_i).wait()
            pltpu.make_async_copy(dst_h.at[pl.ds(base, BLK)],
                                  dstv.at[buf], sem_i).wait()

        def fire_gathers(buf):
            pltpu.async_copy(tab_h.at[srcv.at[buf]], rows.at[buf], sem_g)

        def drain_gathers(buf):
            pltpu.make_async_copy(tab_h.at[srcv.at[buf]], rows.at[buf],
                                  sem_g).wait()

        def fire_scatters(buf):
            pltpu.async_copy(rows.at[buf], shared.at[dstv.at[buf]], sem_s,
                             add=True)

        def drain_scatters(buf):
            pltpu.make_async_copy(rows.at[buf], shared.at[dstv.at[buf]],
                                  sem_s).wait()

        fire_idx(0, 0)
        drain_idx(0, 0)
        fire_gathers(0)

        def block_body(b, carry):
            cur = b % 2
            nxt = 1 - cur

            @pl.when(b > 0)
            def _():
                drain_scatters(nxt)

            @pl.when(b + 1 < nblk)
            def _():
                fire_idx(nxt, b + 1)

            drain_gathers(cur)
            fire_scatters(cur)

            @pl.when(b + 1 < nblk)
            def _():
                drain_idx(nxt, b + 1)
                fire_gathers(nxt)

            return carry

        lax.fori_loop(0, nblk, block_body, 0)
        drain_scatters(0)
        plsc.subcore_barrier()
        pltpu.sync_copy(shared.at[pl.ds(s * TILE_ROWS, TILE_ROWS)],
                        out_h.at[c, pl.ds(s * TILE_ROWS, TILE_ROWS)])

    return k(src_flat, dst_flat, table, zeros32)


def _tc_pre(x_pad, W_pre, b_pre):
    def body(x_ref, w_ref, b_ref, obf_ref, oa_ref, ob_ref):
        h = jnp.dot(x_ref[...], w_ref[...],
                    preferred_element_type=jnp.float32) + b_ref[...]
        h = jnp.maximum(h, 0.0)
        obf_ref[...] = h.astype(jnp.bfloat16)
        oa_ref[...] = h[:, :16]
        ob_ref[...] = h[:, 16:]

    return pl.pallas_call(
        body,
        grid=(NGRID,),
        in_specs=[
            pl.BlockSpec((NODE_BLK, 5), lambda i: (i, 0)),
            pl.BlockSpec((5, 32), lambda i: (0, 0)),
            pl.BlockSpec((1, 32), lambda i: (0, 0)),
        ],
        out_specs=[
            pl.BlockSpec((NODE_BLK, 32), lambda i: (i, 0)),
            pl.BlockSpec((NODE_BLK, 16), lambda i: (i, 0)),
            pl.BlockSpec((NODE_BLK, 16), lambda i: (i, 0)),
        ],
        out_shape=[
            jax.ShapeDtypeStruct((NPAD, 32), jnp.bfloat16),
            jax.ShapeDtypeStruct((NPAD, 16), jnp.float32),
            jax.ShapeDtypeStruct((NPAD, 16), jnp.float32),
        ],
    )(x_pad, W_pre, b_pre.reshape(1, 32))


def _tc_sage(parts, hchunks, cnts, Wl, bl, Wr, din, dout):
    npart = din // 32
    nc = din // 16
    nco = dout // 16
    nbo = dout // 32

    def body(*refs):
        part_refs = refs[:npart]
        h_refs = refs[npart:npart + nc]
        cnt_ref = refs[npart + nc]
        wl_ref, bl_ref, wr_ref = refs[npart + nc + 1: npart + nc + 4]
        out_refs = refs[npart + nc + 4:]

        cnt = jnp.sum(cnt_ref[...], axis=0)
        invc = 1.0 / jnp.clip(cnt, 1.0, None)
        wl = wl_ref[...]
        wr = wr_ref[...]
        acc = jnp.zeros((NODE_BLK, dout), jnp.float32)
        hacc = jnp.zeros((NODE_BLK, dout), jnp.float32)
        for q in range(npart):
            psum = (part_refs[q][0].astype(jnp.float32)
                    + part_refs[q][1].astype(jnp.float32))
            acc = acc + jnp.dot(psum, wl[q * 32:(q + 1) * 32, :],
                                preferred_element_type=jnp.float32)
        for q in range(nc):
            hacc = hacc + jnp.dot(h_refs[q][...], wr[q * 16:(q + 1) * 16, :],
                                  preferred_element_type=jnp.float32)
        lin = acc * invc[:, None] + bl_ref[...] + hacc
        norm = jnp.sqrt(jnp.sum(lin * lin, axis=-1, keepdims=True))
        out = lin / jnp.maximum(norm, 1e-12)
        out = jnp.maximum(out, 0.0)
        for q in range(nbo):
            out_refs[q][...] = out[:, q * 32:(q + 1) * 32].astype(jnp.bfloat16)
        for q in range(nco):
            out_refs[nbo + q][...] = out[:, q * 16:(q + 1) * 16]

    in_specs = (
        [pl.BlockSpec((2, NODE_BLK, 32), lambda i: (0, i, 0))] * npart
        + [pl.BlockSpec((NODE_BLK, 16), lambda i: (i, 0))] * nc
        + [pl.BlockSpec((NW, NODE_BLK), lambda i: (0, i)),
           pl.BlockSpec((din, dout), lambda i: (0, 0)),
           pl.BlockSpec((1, dout), lambda i: (0, 0)),
           pl.BlockSpec((din, dout), lambda i: (0, 0))]
    )
    return pl.pallas_call(
        body,
        grid=(NGRID,),
        in_specs=in_specs,
        out_specs=([pl.BlockSpec((NODE_BLK, 32), lambda i: (i, 0))] * nbo
                   + [pl.BlockSpec((NODE_BLK, 16), lambda i: (i, 0))] * nco),
        out_shape=([jax.ShapeDtypeStruct((NPAD, 32), jnp.bfloat16)] * nbo
                   + [jax.ShapeDtypeStruct((NPAD, 16), jnp.float32)] * nco),
    )(*parts, *hchunks, cnts, Wl, bl.reshape(1, dout), Wr)


def _tc_sage_pool(parts, hchunks, cnts, Wl, bl, Wr, batch3d, din, dout):
    npart = din // 32
    nc = din // 16

    def body(*refs):
        part_refs = refs[:npart]
        h_refs = refs[npart:npart + nc]
        cnt_ref = refs[npart + nc]
        wl_ref, bl_ref, wr_ref, batch_ref = refs[npart + nc + 1:
                                                 npart + nc + 5]
        gsum_ref, gcnt_ref = refs[npart + nc + 5:]

        i = pl.program_id(0)
        cnt = jnp.sum(cnt_ref[...], axis=0)
        invc = 1.0 / jnp.clip(cnt, 1.0, None)
        wl = wl_ref[...]
        wr = wr_ref[...]
        acc = jnp.zeros((NODE_BLK, dout), jnp.float32)
        hacc = jnp.zeros((NODE_BLK, dout), jnp.float32)
        for q in range(npart):
            psum = (part_refs[q][0].astype(jnp.float32)
                    + part_refs[q][1].astype(jnp.float32))
            acc = acc + jnp.dot(psum, wl[q * 32:(q + 1) * 32, :],
                                preferred_element_type=jnp.float32)
        for q in range(nc):
            hacc = hacc + jnp.dot(h_refs[q][...], wr[q * 16:(q + 1) * 16, :],
                                  preferred_element_type=jnp.float32)
        lin = acc * invc[:, None] + bl_ref[...] + hacc
        norm = jnp.sqrt(jnp.sum(lin * lin, axis=-1, keepdims=True))
        h2 = jnp.maximum(lin / jnp.maximum(norm, 1e-12), 0.0)

        bids = batch_ref[0, 0, :]
        onehot = (lax.broadcasted_iota(jnp.int32, (NGRAPH, NODE_BLK), 0)
                  == bids[None, :]).astype(jnp.float32)
        part_gsum = jnp.dot(onehot, h2, preferred_element_type=jnp.float32)
        part_gcnt = jnp.sum(onehot, axis=1, keepdims=True)

        @pl.when(i == 0)
        def _():
            gsum_ref[...] = jnp.zeros_like(gsum_ref)
            gcnt_ref[...] = jnp.zeros_like(gcnt_ref)

        gsum_ref[...] += part_gsum
        gcnt_ref[...] += part_gcnt

    in_specs = (
        [pl.BlockSpec((2, NODE_BLK, 32), lambda i: (0, i, 0))] * npart
        + [pl.BlockSpec((NODE_BLK, 16), lambda i: (i, 0))] * nc
        + [pl.BlockSpec((NW, NODE_BLK), lambda i: (0, i)),
           pl.BlockSpec((din, dout), lambda i: (0, 0)),
           pl.BlockSpec((1, dout), lambda i: (0, 0)),
           pl.BlockSpec((din, dout), lambda i: (0, 0)),
           pl.BlockSpec((1, 1, NODE_BLK), lambda i: (i, 0, 0))]
    )
    return pl.pallas_call(
        body,
        grid=(NGRID,),
        in_specs=in_specs,
        out_specs=[pl.BlockSpec((NGRAPH, dout), lambda i: (0, 0)),
                   pl.BlockSpec((NGRAPH, 1), lambda i: (0, 0))],
        out_shape=[jax.ShapeDtypeStruct((NGRAPH, dout), jnp.float32),
                   jax.ShapeDtypeStruct((NGRAPH, 1), jnp.float32)],
    )(*parts, *hchunks, cnts, Wl, bl.reshape(1, dout), Wr, batch3d)


def _tc_head(gsum, gcnt, Wp1, bp1, Wp2, bp2, Wo, bo):
    def body(gs_ref, gc_ref, w1_ref, b1_ref, w2_ref, b2_ref, wo_ref, bo_ref,
             out_ref):
        g = gs_ref[...] / jnp.clip(gc_ref[...], 1.0, None)
        g = jnp.maximum(jnp.dot(g, w1_ref[...],
                                preferred_element_type=jnp.float32)
                        + b1_ref[...], 0.0)
        g = jnp.maximum(jnp.dot(g, w2_ref[...],
                                preferred_element_type=jnp.float32)
                        + b2_ref[...], 0.0)
        out_ref[...] = jnp.dot(g, wo_ref[...],
                               preferred_element_type=jnp.float32) + bo_ref[...]

    return pl.pallas_call(
        body,
        out_shape=jax.ShapeDtypeStruct((NGRAPH, 1), jnp.float32),
    )(gsum, gcnt, Wp1, bp1.reshape(1, 64), Wp2, bp2.reshape(1, 16),
      Wo, bo.reshape(1, 1))


def kernel(x, edge_index, batch, W_pre, b_pre, Wl1, bl1, Wr1, Wl2, bl2, Wr2,
           Wp1, bp1, Wp2, bp2, Wo, bo):
    src = edge_index[0]
    dst = edge_index[1]
    pad_e = EPAD - E
    src_pad = jnp.concatenate([src, jnp.zeros((pad_e,), jnp.int32)])
    dst_pad = jnp.concatenate([dst, jnp.full((pad_e,), N, jnp.int32)])
    x_pad = jnp.pad(x, ((0, NPAD - N), (0, 0)))
    batch3d = jnp.concatenate(
        [batch, jnp.full((NPAD - N,), NGRAPH, jnp.int32)]).reshape(
            NGRID, 1, NODE_BLK)
    zeros32 = jnp.zeros((NPAD, 32), jnp.bfloat16)

    cnts = _sc_counts(dst_pad)

    h0bf, h0a, h0b = _tc_pre(x_pad, W_pre, b_pre)

    p1 = _sc_segsum32bf(src_pad, dst_pad, h0bf, zeros32)
    h1bf_a, h1bf_b, h1a, h1b, h1c, h1d = _tc_sage(
        [p1], [h0a, h0b], cnts, Wl1, bl1, Wr1, 32, 64)

    p2a = _sc_segsum32bf(src_pad, dst_pad, h1bf_a, zeros32)
    p2b = _sc_segsum32bf(src_pad, dst_pad, h1bf_b, zeros32)
    gsum, gcnt = _tc_sage_pool([p2a, p2b], [h1a, h1b, h1c, h1d], cnts,
                               Wl2, bl2, Wr2, batch3d, 64, 64)

    out = _tc_head(gsum, gcnt, Wp1, bp1, Wp2, bp2, Wo, bo)
    return out[:, 0]


# speedup vs baseline: 1.0701x; 1.0701x over previous
import functools

import jax
import jax.numpy as jnp
from jax import lax
from jax.experimental import pallas as pl
from jax.experimental.pallas import tpu as pltpu
from jax.experimental.pallas import tpu_sc as plsc

N = 100000
NPAD = 100352
E = 3200000
EPAD = 3276800
NW = 32
EW = EPAD // NW
BLK = 640
NBLK = EW // BLK
ROWS_PER_W = EW // 128
NGRAPH = 128
NODE_BLK = 1024
NBLK_FAST = 289
NBLK_SLOW = 31
NGRID = NPAD // NODE_BLK
TILE_ROWS = NPAD // 16


def _sc_counts(dst_flat):
    mesh = plsc.VectorSubcoreMesh(core_axis_name="c", subcore_axis_name="s")

    @functools.partial(
        pl.kernel,
        out_type=jax.ShapeDtypeStruct((NW, NPAD), jnp.float32),
        scratch_types=[
            pltpu.VMEM((BLK,), jnp.int32),
            pltpu.VMEM((NPAD,), jnp.float32),
        ],
        mesh=mesh,
        compiler_params=pltpu.CompilerParams(use_tc_tiling_on_sc=False,
                                             needs_layout_passes=False),
    )
    def k(dst_h, out_h, dstbuf, cnt):
        c = lax.axis_index("c")
        s = lax.axis_index("s")
        w = s * 2 + c

        def zero_body(j, carry):
            cnt[pl.ds(j * 16, 16)] = jnp.zeros((16,), jnp.float32)
            return carry

        lax.fori_loop(0, NPAD // 16, zero_body, 0)

        ones = jnp.ones((16,), jnp.float32)

        def block_body(b, carry):
            base = w * EW + b * BLK
            pltpu.sync_copy(dst_h.at[pl.ds(base, BLK)], dstbuf)

            def inner(k2, carry2):
                idx = dstbuf[pl.ds(k2 * 16, 16)]
                plsc.addupdate_scatter(cnt, [idx], ones)
                return carry2

            lax.fori_loop(0, BLK // 16, inner, 0)
            return carry

        lax.fori_loop(0, NBLK, block_body, 0)
        pltpu.sync_copy(cnt, out_h.at[w])

    return k(dst_flat)


def _sc_segsum32bf(src_flat, dst_flat, table, zeros32):
    mesh = plsc.VectorSubcoreMesh(core_axis_name="c", subcore_axis_name="s")

    @functools.partial(
        pl.kernel,
        out_type=jax.ShapeDtypeStruct((2, NPAD, 32), jnp.bfloat16),
        scratch_types=[
            pltpu.VMEM((2, BLK), jnp.int32),
            pltpu.VMEM((2, BLK), jnp.int32),
            pltpu.VMEM((2, BLK, 32), jnp.bfloat16),
            pltpu.VMEM_SHARED((NPAD, 32), jnp.bfloat16),
            pltpu.SemaphoreType.DMA,
            pltpu.SemaphoreType.DMA,
            pltpu.SemaphoreType.DMA,
        ],
        mesh=mesh,
        compiler_params=pltpu.CompilerParams(use_tc_tiling_on_sc=False),
    )
    def k(src_h, dst_h, tab_h, z_h, out_h, srcv, dstv, rows, shared,
          sem_i, sem_g, sem_s):
        c = lax.axis_index("c")
        s = lax.axis_index("s")

        pltpu.sync_copy(z_h.at[pl.ds(s * TILE_ROWS, TILE_ROWS)],
                        shared.at[pl.ds(s * TILE_ROWS, TILE_ROWS)])
        plsc.subcore_barrier()

        nblk = jnp.where(c == 0, NBLK_FAST, NBLK_SLOW)
        sc_base = jnp.where(c == 0, 0, NBLK_FAST * 16 * BLK)
        tile_base = sc_base + s * nblk * BLK

        def fire_idx(buf, b):
            base = tile_base + b * BLK
            pltpu.async_copy(src_h.at[pl.ds(base, BLK)], srcv.at[buf],
                             sem_i)
            pltpu.async_copy(dst_h.at[pl.ds(base, BLK)], dstv.at[buf],
                             sem_i)

        def drain_idx(buf, b):
            base = tile_base + b * BLK
            pltpu.make_async_copy(src_h.at[pl.ds(base, BLK)],
                                  srcv.at[buf], sem_i).wait()
            pltpu.make_async_copy(dst_h.at[pl.ds(base, BLK)],
                                  dstv.at[buf], sem_i).wait()

        def fire_gathers(buf):
            pltpu.async_copy(tab_h.at[srcv.at[buf]], rows.at[buf], sem_g)

        def drain_gathers(buf):
            pltpu.make_async_copy(tab_h.at[srcv.at[buf]], rows.at[buf],
                                  sem_g).wait()

        def fire_scatters(buf):
            pltpu.async_copy(rows.at[buf], shared.at[dstv.at[buf]], sem_s,
                             add=True)

        def drain_scatters(buf):
            pltpu.make_async_copy(rows.at[buf], shared.at[dstv.at[buf]],
                                  sem_s).wait()

        fire_idx(0, 0)
        drain_idx(0, 0)
        fire_gathers(0)

        def block_body(b, carry):
            cur = b % 2
            nxt = 1 - cur

            @pl.when(b > 0)
            def _():
                drain_scatters(nxt)

            @pl.when(b + 1 < nblk)
            def _():
                fire_idx(nxt, b + 1)

            drain_gathers(cur)
            fire_scatters(cur)

            @pl.when(b + 1 < nblk)
            def _():
                drain_idx(nxt, b + 1)
                fire_gathers(nxt)

            return carry

        lax.fori_loop(0, nblk, block_body, 0)
        drain_scatters(0)
        plsc.subcore_barrier()
        pltpu.sync_copy(shared.at[pl.ds(s * TILE_ROWS, TILE_ROWS)],
                        out_h.at[c, pl.ds(s * TILE_ROWS, TILE_ROWS)])

    return k(src_flat, dst_flat, table, zeros32)


def _tc_pre(x_pad, W_pre, b_pre):
    def body(x_ref, w_ref, b_ref, obf_ref, oa_ref, ob_ref):
        h = jnp.dot(x_ref[...], w_ref[...],
                    preferred_element_type=jnp.float32) + b_ref[...]
        h = jnp.maximum(h, 0.0)
        obf_ref[...] = h.astype(jnp.bfloat16)
        oa_ref[...] = h[:, :16]
        ob_ref[...] = h[:, 16:]

    return pl.pallas_call(
        body,
        grid=(NGRID,),
        in_specs=[
            pl.BlockSpec((NODE_BLK, 5), lambda i: (i, 0)),
            pl.BlockSpec((5, 32), lambda i: (0, 0)),
            pl.BlockSpec((1, 32), lambda i: (0, 0)),
        ],
        out_specs=[
            pl.BlockSpec((NODE_BLK, 32), lambda i: (i, 0)),
            pl.BlockSpec((NODE_BLK, 16), lambda i: (i, 0)),
            pl.BlockSpec((NODE_BLK, 16), lambda i: (i, 0)),
        ],
        out_shape=[
            jax.ShapeDtypeStruct((NPAD, 32), jnp.bfloat16),
            jax.ShapeDtypeStruct((NPAD, 16), jnp.float32),
            jax.ShapeDtypeStruct((NPAD, 16), jnp.float32),
        ],
    )(x_pad, W_pre, b_pre.reshape(1, 32))


def _tc_sage(parts, hchunks, cnts, Wl, bl, Wr, din, dout):
    npart = din // 32
    nc = din // 16
    nco = dout // 16
    nbo = dout // 32

    def body(*refs):
        part_refs = refs[:npart]
        h_refs = refs[npart:npart + nc]
        cnt_ref = refs[npart + nc]
        wl_ref, bl_ref, wr_ref = refs[npart + nc + 1: npart + nc + 4]
        out_refs = refs[npart + nc + 4:]

        cnt = jnp.sum(cnt_ref[...], axis=0)
        invc = 1.0 / jnp.clip(cnt, 1.0, None)
        wl = wl_ref[...]
        wr = wr_ref[...]
        acc = jnp.zeros((NODE_BLK, dout), jnp.float32)
        hacc = jnp.zeros((NODE_BLK, dout), jnp.float32)
        for q in range(npart):
            psum = (part_refs[q][0].astype(jnp.float32)
                    + part_refs[q][1].astype(jnp.float32))
            acc = acc + jnp.dot(psum, wl[q * 32:(q + 1) * 32, :],
                                preferred_element_type=jnp.float32)
        for q in range(nc):
            hacc = hacc + jnp.dot(h_refs[q][...], wr[q * 16:(q + 1) * 16, :],
                                  preferred_element_type=jnp.float32)
        lin = acc * invc[:, None] + bl_ref[...] + hacc
        norm = jnp.sqrt(jnp.sum(lin * lin, axis=-1, keepdims=True))
        out = lin / jnp.maximum(norm, 1e-12)
        out = jnp.maximum(out, 0.0)
        for q in range(nbo):
            out_refs[q][...] = out[:, q * 32:(q + 1) * 32].astype(jnp.bfloat16)
        for q in range(nco):
            out_refs[nbo + q][...] = out[:, q * 16:(q + 1) * 16]

    in_specs = (
        [pl.BlockSpec((2, NODE_BLK, 32), lambda i: (0, i, 0))] * npart
        + [pl.BlockSpec((NODE_BLK, 16), lambda i: (i, 0))] * nc
        + [pl.BlockSpec((NW, NODE_BLK), lambda i: (0, i)),
           pl.BlockSpec((din, dout), lambda i: (0, 0)),
           pl.BlockSpec((1, dout), lambda i: (0, 0)),
           pl.BlockSpec((din, dout), lambda i: (0, 0))]
    )
    return pl.pallas_call(
        body,
        grid=(NGRID,),
        in_specs=in_specs,
        out_specs=([pl.BlockSpec((NODE_BLK, 32), lambda i: (i, 0))] * nbo
                   + [pl.BlockSpec((NODE_BLK, 16), lambda i: (i, 0))] * nco),
        out_shape=([jax.ShapeDtypeStruct((NPAD, 32), jnp.bfloat16)] * nbo
                   + [jax.ShapeDtypeStruct((NPAD, 16), jnp.float32)] * nco),
    )(*parts, *hchunks, cnts, Wl, bl.reshape(1, dout), Wr)


def _tc_sage_pool(parts, hchunks, cnts, Wl, bl, Wr, batch3d, din, dout):
    npart = din // 32
    nc = din // 16

    def body(*refs):
        part_refs = refs[:npart]
        h_refs = refs[npart:npart + nc]
        cnt_ref = refs[npart + nc]
        wl_ref, bl_ref, wr_ref, batch_ref = refs[npart + nc + 1:
                                                 npart + nc + 5]
        gsum_ref, gcnt_ref = refs[npart + nc + 5:]

        i = pl.program_id(0)
        cnt = jnp.sum(cnt_ref[...], axis=0)
        invc = 1.0 / jnp.clip(cnt, 1.0, None)
        wl = wl_ref[...]
        wr = wr_ref[...]
        acc = jnp.zeros((NODE_BLK, dout), jnp.float32)
        hacc = jnp.zeros((NODE_BLK, dout), jnp.float32)
        for q in range(npart):
            psum = (part_refs[q][0].astype(jnp.float32)
                    + part_refs[q][1].astype(jnp.float32))
            acc = acc + jnp.dot(psum, wl[q * 32:(q + 1) * 32, :],
                                preferred_element_type=jnp.float32)
        for q in range(nc):
            hacc = hacc + jnp.dot(h_refs[q][...], wr[q * 16:(q + 1) * 16, :],
                                  preferred_element_type=jnp.float32)
        lin = acc * invc[:, None] + bl_ref[...] + hacc
        norm = jnp.sqrt(jnp.sum(lin * lin, axis=-1, keepdims=True))
        h2 = jnp.maximum(lin / jnp.maximum(norm, 1e-12), 0.0)

        bids = batch_ref[0, 0, :]
        onehot = (lax.broadcasted_iota(jnp.int32, (NGRAPH, NODE_BLK), 0)
                  == bids[None, :]).astype(jnp.float32)
        part_gsum = jnp.dot(onehot, h2, preferred_element_type=jnp.float32)
        part_gcnt = jnp.sum(onehot, axis=1, keepdims=True)

        @pl.when(i == 0)
        def _():
            gsum_ref[...] = jnp.zeros_like(gsum_ref)
            gcnt_ref[...] = jnp.zeros_like(gcnt_ref)

        gsum_ref[...] += part_gsum
        gcnt_ref[...] += part_gcnt

    in_specs = (
        [pl.BlockSpec((2, NODE_BLK, 32), lambda i: (0, i, 0))] * npart
        + [pl.BlockSpec((NODE_BLK, 16), lambda i: (i, 0))] * nc
        + [pl.BlockSpec((NW, NODE_BLK), lambda i: (0, i)),
           pl.BlockSpec((din, dout), lambda i: (0, 0)),
           pl.BlockSpec((1, dout), lambda i: (0, 0)),
           pl.BlockSpec((din, dout), lambda i: (0, 0)),
           pl.BlockSpec((1, 1, NODE_BLK), lambda i: (i, 0, 0))]
    )
    return pl.pallas_call(
        body,
        grid=(NGRID,),
        in_specs=in_specs,
        out_specs=[pl.BlockSpec((NGRAPH, dout), lambda i: (0, 0)),
                   pl.BlockSpec((NGRAPH, 1), lambda i: (0, 0))],
        out_shape=[jax.ShapeDtypeStruct((NGRAPH, dout), jnp.float32),
                   jax.ShapeDtypeStruct((NGRAPH, 1), jnp.float32)],
    )(*parts, *hchunks, cnts, Wl, bl.reshape(1, dout), Wr, batch3d)


def _tc_head(gsum, gcnt, Wp1, bp1, Wp2, bp2, Wo, bo):
    def body(gs_ref, gc_ref, w1_ref, b1_ref, w2_ref, b2_ref, wo_ref, bo_ref,
             out_ref):
        g = gs_ref[...] / jnp.clip(gc_ref[...], 1.0, None)
        g = jnp.maximum(jnp.dot(g, w1_ref[...],
                                preferred_element_type=jnp.float32)
                        + b1_ref[...], 0.0)
        g = jnp.maximum(jnp.dot(g, w2_ref[...],
                                preferred_element_type=jnp.float32)
                        + b2_ref[...], 0.0)
        out_ref[...] = jnp.dot(g, wo_ref[...],
                               preferred_element_type=jnp.float32) + bo_ref[...]

    return pl.pallas_call(
        body,
        out_shape=jax.ShapeDtypeStruct((NGRAPH, 1), jnp.float32),
    )(gsum, gcnt, Wp1, bp1.reshape(1, 64), Wp2, bp2.reshape(1, 16),
      Wo, bo.reshape(1, 1))


def kernel(x, edge_index, batch, W_pre, b_pre, Wl1, bl1, Wr1, Wl2, bl2, Wr2,
           Wp1, bp1, Wp2, bp2, Wo, bo):
    src = edge_index[0]
    dst = edge_index[1]
    pad_e = EPAD - E
    src_pad = jnp.concatenate([src, jnp.zeros((pad_e,), jnp.int32)])
    dst_pad = jnp.concatenate([dst, jnp.full((pad_e,), N, jnp.int32)])
    x_pad = jnp.pad(x, ((0, NPAD - N), (0, 0)))
    batch3d = jnp.concatenate(
        [batch, jnp.full((NPAD - N,), NGRAPH, jnp.int32)]).reshape(
            NGRID, 1, NODE_BLK)
    zeros32 = jnp.zeros((NPAD, 32), jnp.bfloat16)

    cnts = _sc_counts(dst_pad)

    h0bf, h0a, h0b = _tc_pre(x_pad, W_pre, b_pre)

    p1 = _sc_segsum32bf(src_pad, dst_pad, h0bf, zeros32)
    h1bf_a, h1bf_b, h1a, h1b, h1c, h1d = _tc_sage(
        [p1], [h0a, h0b], cnts, Wl1, bl1, Wr1, 32, 64)

    p2a = _sc_segsum32bf(src_pad, dst_pad, h1bf_a, zeros32)
    p2b = _sc_segsum32bf(src_pad, dst_pad, h1bf_b, zeros32)
    gsum, gcnt = _tc_sage_pool([p2a, p2b], [h1a, h1b, h1c, h1d], cnts,
                               Wl2, bl2, Wr2, batch3d, 64, 64)

    out = _tc_head(gsum, gcnt, Wp1, bp1, Wp2, bp2, Wo, bo)
    return out[:, 0]
